# baseline (speedup 1.0000x reference)
import functools

import jax
import jax.numpy as jnp
from jax import lax
from jax.experimental import pallas as pl
from jax.experimental.pallas import tpu as pltpu
from jax.experimental.pallas import tpu_sc as plsc

N = 10000
D = 128
G = 64
C = 10
E = 320000
K = 64
NB = 3
CPT = 160
NCHUNK = 32 * CPT
E_PAD = NCHUNK * K
DEGW = 128
NDEG = E_PAD // DEGW
N_A = 10112
RPT = N_A // 16
BN = 400
GRID = N // BN


def _deg_body(dst_hbm, deg_hbm, idx_v, deg_v):
    c = lax.axis_index("c")
    s = lax.axis_index("s")

    @pl.when(s == 0)
    def _():
        def zero(i, carry):
            deg_v[pl.ds(i * 16, 16)] = jnp.zeros((16,), jnp.float32)
            return carry

        lax.fori_loop(0, N_A // 16, zero, 0)

        def outer(k, carry):
            base = pl.multiple_of(c * (NDEG // 2) + k * 128, 8)
            pltpu.sync_copy(dst_hbm.at[pl.ds(base, 128)], idx_v)

            def row(r, carry2):
                def sub(q, carry3):
                    idx = idx_v[r, pl.ds(q * 16, 16)]
                    plsc.addupdate_scatter(deg_v, [idx],
                                           jnp.ones((16,), jnp.float32))
                    return carry3

                return lax.fori_loop(0, 8, sub, carry2)

            lax.fori_loop(0, 128, row, 0)
            return carry

        lax.fori_loop(0, NDEG // 2 // 128, outer, 0)
        pltpu.sync_copy(deg_v, deg_hbm.at[c])


def _scatter_body(g_hbm, src_hbm, dst_hbm, out_hbm, *refs):
    src_v, dst_v = refs[0], refs[1]
    rows = refs[2:2 + NB]
    acc_sh = refs[2 + NB]
    gsems = refs[3 + NB:3 + 2 * NB]
    ssems = refs[3 + 2 * NB:3 + 3 * NB]
    c = lax.axis_index("c")
    s = lax.axis_index("s")
    base = pl.multiple_of(c * (NCHUNK // 2) + s * CPT, 8)
    rows0_v = rows[0]

    def zero(i, carry):
        r = i // 8
        q = i - r * 8
        rows0_v[r, pl.ds(q * 16, 16)] = jnp.zeros((16,), jnp.float32)
        return carry

    lax.fori_loop(0, K * (D // 16), zero, 0)

    row0 = pl.multiple_of(s * RPT, 8)
    for kk in range(RPT // K):
        pltpu.sync_copy(rows0_v, acc_sh.at[pl.ds(row0 + kk * K, K)])
    rem = RPT - (RPT // K) * K
    pltpu.sync_copy(rows0_v.at[pl.ds(0, rem)],
                    acc_sh.at[pl.ds(row0 + (RPT // K) * K, rem)])

    pltpu.sync_copy(src_hbm.at[pl.ds(base, CPT)], src_v)
    pltpu.sync_copy(dst_hbm.at[pl.ds(base, CPT)], dst_v)

    plsc.subcore_barrier()

    def src_at(j):
        return src_v.at[j]

    def dst_at(j):
        return dst_v.at[j]

    for r in range(NB - 1):
        pltpu.async_copy(g_hbm.at[src_at(r)], rows[r], gsems[r])

    def ring(jj, carry):
        for r in range(NB):
            j = jj * NB + r
            rp = (r + NB - 1) % NB

            @pl.when(j > 0)
            def _():
                pltpu.make_async_copy(rows[rp], acc_sh.at[dst_at(j - 1)],
                                      ssems[rp]).wait()

            @pl.when(j + NB - 1 < CPT)
            def _():
                pltpu.async_copy(g_hbm.at[src_at(j + NB - 1)], rows[rp],
                                 gsems[rp])

            pltpu.make_async_copy(g_hbm.at[src_at(j)],
                                  rows[r], gsems[r]).wait()
            pltpu.async_copy(rows[r], acc_sh.at[dst_at(j)], ssems[r],
                             add=True)
        return carry

    nmain = (CPT // NB) * NB
    lax.fori_loop(0, CPT // NB, ring, 0)
    for j in range(nmain, CPT):
        r = j % NB
        rp = (r + NB - 1) % NB
        pltpu.make_async_copy(rows[rp], acc_sh.at[dst_at(j - 1)],
                              ssems[rp]).wait()
        pltpu.make_async_copy(g_hbm.at[src_at(j)], rows[r], gsems[r]).wait()
        pltpu.async_copy(rows[r], acc_sh.at[dst_at(j)], ssems[r], add=True)
    pltpu.make_async_copy(rows[(CPT - 1) % NB],
                          acc_sh.at[dst_at(CPT - 1)],
                          ssems[(CPT - 1) % NB]).wait()

    plsc.subcore_barrier()
    pltpu.sync_copy(acc_sh.at[pl.ds(row0, RPT)],
                    out_hbm.at[c, pl.ds(row0, RPT)])


@functools.cache
def _sc_kernels():
    mesh = plsc.VectorSubcoreMesh(core_axis_name="c", subcore_axis_name="s")
    deg = pl.kernel(
        _deg_body,
        out_type=jax.ShapeDtypeStruct((2, N_A), jnp.float32),
        mesh=mesh,
        compiler_params=pltpu.CompilerParams(needs_layout_passes=False),
        scratch_types=[
            pltpu.VMEM((128, DEGW), jnp.int32),
            pltpu.VMEM((N_A,), jnp.float32),
        ],
    )
    scatter = pl.kernel(
        _scatter_body,
        out_type=jax.ShapeDtypeStruct((2, N_A, D), jnp.float32),
        mesh=mesh,
        compiler_params=pltpu.CompilerParams(use_tc_tiling_on_sc=False),
        scratch_types=[
            pltpu.VMEM((CPT, K), jnp.int32),
            pltpu.VMEM((CPT, K), jnp.int32),
        ] + [pltpu.VMEM((K, D), jnp.float32)] * NB + [
            pltpu.VMEM_SHARED((N_A, D), jnp.float32),
        ] + [pltpu.SemaphoreType.DMA] * (2 * NB),
    )
    return deg, scatter


def _k1_body(deg_ref, x_ref, w_ref, dinv_ref, g_ref):
    dinv = lax.rsqrt(1.0 + deg_ref[0] + deg_ref[1])
    dinv_ref[...] = dinv
    g_ref[...] = jnp.dot(x_ref[...], w_ref[...],
                         preferred_element_type=jnp.float32) * dinv


def _klayer_body(a_ref, g_ref, dinv_ref, w_ref, gout_ref):
    dinv = dinv_ref[...]
    xl = jnp.maximum(dinv * (a_ref[0] + a_ref[1] + g_ref[...]), 0.0)
    gout_ref[...] = jnp.dot(xl, w_ref[...],
                            preferred_element_type=jnp.float32) * dinv


def _k5_body(a_ref, g_ref, dinv_ref, fcw_ref, fcb_ref, batch_ref,
             logp_ref, xr_ref):
    dinv = dinv_ref[...]
    x4 = jnp.maximum(dinv * (a_ref[0] + a_ref[1] + g_ref[...]), 0.0)
    logits = jnp.dot(x4, fcw_ref[...],
                     preferred_element_type=jnp.float32) + fcb_ref[...]
    m = jnp.max(logits, axis=1, keepdims=True)
    lse = jnp.log(jnp.sum(jnp.exp(logits - m), axis=1, keepdims=True)) + m
    logp_ref[...] = (logits - lse)[:, :C]

    oh = (batch_ref[0] == lax.broadcasted_iota(jnp.int32, (G, 1), 0)
          ).astype(jnp.float32)
    contrib = jnp.dot(oh, x4, preferred_element_type=jnp.float32)

    @pl.when(pl.program_id(0) == 0)
    def _():
        xr_ref[...] = contrib

    @pl.when(pl.program_id(0) > 0)
    def _():
        xr_ref[...] = xr_ref[...] + contrib


_a_spec = pl.BlockSpec((2, BN, D), lambda i: (0, i, 0))
_g_spec = pl.BlockSpec((BN, D), lambda i: (i, 0))
_dinv_spec = pl.BlockSpec((BN, 1), lambda i: (i, 0))
_w_spec = pl.BlockSpec((D, D), lambda i: (0, 0))
_g_shape = jax.ShapeDtypeStruct((N, D), jnp.float32)


@functools.cache
def _tc_calls(interpret=False):
    k1 = pl.pallas_call(
        _k1_body,
        interpret=interpret,
        grid=(GRID,),
        in_specs=[
            pl.BlockSpec((2, BN, 1), lambda i: (0, i, 0)),
            _g_spec,
            _w_spec,
        ],
        out_specs=[
            _dinv_spec,
            _g_spec,
        ],
        out_shape=[
            jax.ShapeDtypeStruct((N, 1), jnp.float32),
            _g_shape,
        ],
    )

    klayer = pl.pallas_call(
        _klayer_body,
        interpret=interpret,
        grid=(GRID,),
        in_specs=[
            _a_spec,
            _g_spec,
            _dinv_spec,
            _w_spec,
        ],
        out_specs=_g_spec,
        out_shape=_g_shape,
    )

    k5 = pl.pallas_call(
        _k5_body,
        interpret=interpret,
        grid=(GRID,),
        in_specs=[
            _a_spec,
            _g_spec,
            _dinv_spec,
            _w_spec,
            pl.BlockSpec((1, D), lambda i: (0, 0)),
            pl.BlockSpec((1, 1, BN), lambda i: (i, 0, 0)),
        ],
        out_specs=[
            pl.BlockSpec((BN, C), lambda i: (i, 0)),
            pl.BlockSpec((G, D), lambda i: (0, 0)),
        ],
        out_shape=[
            jax.ShapeDtypeStruct((N, C), jnp.float32),
            jax.ShapeDtypeStruct((G, D), jnp.float32),
        ],
    )
    return k1, klayer, k5


def kernel(x, edge_index, batch, W0, W1, W2, W3, fc_w, fc_b):
    pad = E_PAD - E
    src_flat = jnp.concatenate([edge_index[0], jnp.zeros((pad,), jnp.int32)])
    dst_flat = jnp.concatenate([edge_index[1], jnp.full((pad,), N, jnp.int32)])
    src2 = src_flat.reshape(NCHUNK, K)
    dst2 = dst_flat.reshape(NCHUNK, K)
    dstd = dst_flat.reshape(NDEG, DEGW)
    _deg_kernel, _scatter_kernel = _sc_kernels()
    _k1_call, _klayer_call, _k5_call = _tc_calls()

    deg = _deg_kernel(dstd)
    deg3 = deg.reshape(2, N_A, 1)
    dinv, g = _k1_call(deg3, x, W0)

    for W in (W1, W2, W3):
        A = _scatter_kernel(g, src2, dst2)
        g = _klayer_call(A, g, dinv, W)
    A = _scatter_kernel(g, src2, dst2)

    fcw_pad = jnp.zeros((D, D), jnp.float32).at[:, :C].set(fc_w)
    fcb_pad = jnp.full((1, D), -1e30, jnp.float32).at[0, :C].set(fc_b)
    batch_row = batch.reshape(GRID, 1, BN)

    logp, xr = _k5_call(A, g, dinv, fcw_pad, fcb_pad, batch_row)
    return logp, xr


# speedup vs baseline: 1.6382x; 1.6382x over previous
import functools

import jax
import jax.numpy as jnp
from jax import lax
from jax.experimental import pallas as pl
from jax.experimental.pallas import tpu as pltpu
from jax.experimental.pallas import tpu_sc as plsc

N = 10000
D = 128
H = D // 2
G = 64
C = 10
E = 320000
K = 128
CHW = 1
NB = 5
CPT = 160
NCHUNK = 16 * CPT
E_PAD = NCHUNK * K
N_A = 10240
RPT = N_A // 16
BN = 400
GRID = N // BN


def _deg_body(dst_hbm, deg_hbm, idx_v, deg_v, sum_v, slab_sh):
    c = lax.axis_index("c")
    s = lax.axis_index("s")
    wid = c * 16 + s

    def zero(i, carry):
        deg_v[pl.ds(i * 16, 16)] = jnp.zeros((16,), jnp.float32)
        return carry

    lax.fori_loop(0, N_A // 16, zero, 0)

    base = pl.multiple_of(wid * (NCHUNK // 32), 8)
    pltpu.sync_copy(dst_hbm.at[pl.ds(base, NCHUNK // 32)], idx_v)

    def row(r, carry2):
        def sub(q, carry3):
            idx = idx_v[r, pl.ds(q * 16, 16)]
            plsc.addupdate_scatter(deg_v, [idx],
                                   jnp.ones((16,), jnp.float32))
            return carry3

        return lax.fori_loop(0, K // 16, sub, carry2)

    lax.fori_loop(0, NCHUNK // 32, row, 0)

    pltpu.sync_copy(deg_v, slab_sh.at[s])
    plsc.subcore_barrier()

    row0 = pl.multiple_of(s * RPT, 8)
    pltpu.sync_copy(slab_sh.at[:, pl.ds(row0, RPT)], sum_v)

    def reduce(q, carry):
        tot = sum_v[0, pl.ds(q * 16, 16)]
        for t in range(1, 16):
            tot = tot + sum_v[t, pl.ds(q * 16, 16)]
        deg_v[pl.ds(q * 16, 16)] = tot
        return carry

    lax.fori_loop(0, RPT // 16, reduce, 0)
    pltpu.sync_copy(deg_v.at[pl.ds(0, RPT)],
                    deg_hbm.at[c, pl.ds(row0, RPT)])


def _scatter_body(g_hbm, src_hbm, dst_hbm, out_hbm, *refs):
    src_v, dst_v = refs[0], refs[1]
    rows = refs[2:2 + NB]
    acc_sh = refs[2 + NB]
    gsems = refs[3 + NB:3 + 2 * NB]
    ssems = refs[3 + 2 * NB:3 + 3 * NB]
    c = lax.axis_index("c")
    s = lax.axis_index("s")
    base = pl.multiple_of(s * CPT, 8)
    rows0_v = rows[0]

    def zero(i, carry):
        r = i // 4
        q = i - r * 4
        rows0_v[r, pl.ds(q * 16, 16)] = jnp.zeros((16,), jnp.float32)
        return carry

    lax.fori_loop(0, K * (H // 16), zero, 0)

    row0 = pl.multiple_of(s * RPT, 8)
    for kk in range(RPT // K):
        pltpu.sync_copy(rows0_v, acc_sh.at[pl.ds(row0 + kk * K, K)])
    rem = RPT - (RPT // K) * K
    if rem:
        pltpu.sync_copy(rows0_v.at[pl.ds(0, rem)],
                        acc_sh.at[pl.ds(row0 + (RPT // K) * K, rem)])

    pltpu.sync_copy(src_hbm.at[pl.ds(base, CPT)], src_v)
    pltpu.sync_copy(dst_hbm.at[pl.ds(base, CPT)], dst_v)

    plsc.subcore_barrier()

    g_half = g_hbm.at[c]
    nstep = CPT

    def src_at(j):
        return src_v.at[j]

    def dst_at(j):
        return dst_v.at[j]

    for r in range(NB - 1):
        pltpu.async_copy(g_half.at[src_at(r)], rows[r], gsems[r])

    def ring(jj, carry):
        for r in range(NB):
            j = jj * NB + r
            rp = (r + NB - 1) % NB

            @pl.when(j > 0)
            def _():
                pltpu.make_async_copy(rows[rp], acc_sh.at[dst_at(j - 1)],
                                      ssems[rp]).wait()

            @pl.when(j + NB - 1 < nstep)
            def _():
                pltpu.async_copy(g_half.at[src_at(j + NB - 1)], rows[rp],
                                 gsems[rp])

            pltpu.make_async_copy(g_half.at[src_at(j)],
                                  rows[r], gsems[r]).wait()
            pltpu.async_copy(rows[r], acc_sh.at[dst_at(j)], ssems[r],
                             add=True)
        return carry

    lax.fori_loop(0, nstep // NB, ring, 0)
    pltpu.make_async_copy(rows[(nstep - 1) % NB],
                          acc_sh.at[dst_at(nstep - 1)],
                          ssems[(nstep - 1) % NB]).wait()

    plsc.subcore_barrier()
    pltpu.sync_copy(acc_sh.at[pl.ds(row0, RPT)],
                    out_hbm.at[c, pl.ds(row0, RPT)])


@functools.cache
def _sc_kernels():
    mesh = plsc.VectorSubcoreMesh(core_axis_name="c", subcore_axis_name="s")
    deg = pl.kernel(
        _deg_body,
        out_type=jax.ShapeDtypeStruct((2, N_A), jnp.float32),
        mesh=mesh,
        compiler_params=pltpu.CompilerParams(needs_layout_passes=False),
        scratch_types=[
            pltpu.VMEM((NCHUNK // 32, K), jnp.int32),
            pltpu.VMEM((N_A,), jnp.float32),
            pltpu.VMEM((16, RPT), jnp.float32),
            pltpu.VMEM_SHARED((16, N_A), jnp.float32),
        ],
    )
    scatter = pl.kernel(
        _scatter_body,
        out_type=jax.ShapeDtypeStruct((2, N_A, H), jnp.float32),
        mesh=mesh,
        compiler_params=pltpu.CompilerParams(use_tc_tiling_on_sc=False),
        scratch_types=[
            pltpu.VMEM((CPT, K), jnp.int32),
            pltpu.VMEM((CPT, K), jnp.int32),
        ] + [pltpu.VMEM((K, H), jnp.float32)] * NB + [
            pltpu.VMEM_SHARED((N_A, H), jnp.float32),
        ] + [pltpu.SemaphoreType.DMA] * (2 * NB),
    )
    return deg, scatter


def _k1_body(deg_ref, x_ref, w_ref, dinv_ref, g_ref):
    dinv = lax.rsqrt(1.0 + deg_ref[0] + deg_ref[1])
    dinv_ref[...] = dinv
    res = jnp.dot(x_ref[...], w_ref[...],
                  preferred_element_type=jnp.float32) * dinv
    g_ref[0] = res[:, :H]
    g_ref[1] = res[:, H:]


def _klayer_body(a_ref, g_ref, dinv_ref, w_ref, gout_ref):
    dinv = dinv_ref[...]
    pre = jnp.concatenate([a_ref[0] + g_ref[0], a_ref[1] + g_ref[1]], axis=1)
    xl = jnp.maximum(dinv * pre, 0.0)
    res = jnp.dot(xl, w_ref[...], preferred_element_type=jnp.float32) * dinv
    gout_ref[0] = res[:, :H]
    gout_ref[1] = res[:, H:]


def _k5_body(a_ref, g_ref, dinv_ref, fcw_ref, fcb_ref, batch_ref,
             logp_ref, xr_ref):
    dinv = dinv_ref[...]
    pre = jnp.concatenate([a_ref[0] + g_ref[0], a_ref[1] + g_ref[1]], axis=1)
    x4 = jnp.maximum(dinv * pre, 0.0)
    logits = jnp.dot(x4, fcw_ref[...],
                     preferred_element_type=jnp.float32) + fcb_ref[...]
    m = jnp.max(logits, axis=1, keepdims=True)
    lse = jnp.log(jnp.sum(jnp.exp(logits - m), axis=1, keepdims=True)) + m
    logp_ref[...] = (logits - lse)[:, :C]

    oh = (batch_ref[0] == lax.broadcasted_iota(jnp.int32, (G, 1), 0)
          ).astype(jnp.float32)
    contrib = jnp.dot(oh, x4, preferred_element_type=jnp.float32)

    @pl.when(pl.program_id(0) == 0)
    def _():
        xr_ref[...] = contrib

    @pl.when(pl.program_id(0) > 0)
    def _():
        xr_ref[...] = xr_ref[...] + contrib


_half_spec = pl.BlockSpec((2, BN, H), lambda i: (0, i, 0))
_dinv_spec = pl.BlockSpec((BN, 1), lambda i: (i, 0))
_w_spec = pl.BlockSpec((D, D), lambda i: (0, 0))
_g_shape = jax.ShapeDtypeStruct((2, N, H), jnp.float32)


@functools.cache
def _tc_calls(interpret=False):
    k1 = pl.pallas_call(
        _k1_body,
        interpret=interpret,
        grid=(GRID,),
        in_specs=[
            pl.BlockSpec((2, BN, 1), lambda i: (0, i, 0)),
            pl.BlockSpec((BN, D), lambda i: (i, 0)),
            _w_spec,
        ],
        out_specs=[
            _dinv_spec,
            _half_spec,
        ],
        out_shape=[
            jax.ShapeDtypeStruct((N, 1), jnp.float32),
            _g_shape,
        ],
    )

    klayer = pl.pallas_call(
        _klayer_body,
        interpret=interpret,
        grid=(GRID,),
        in_specs=[
            _half_spec,
            _half_spec,
            _dinv_spec,
            _w_spec,
        ],
        out_specs=_half_spec,
        out_shape=_g_shape,
    )

    k5 = pl.pallas_call(
        _k5_body,
        interpret=interpret,
        grid=(GRID,),
        in_specs=[
            _half_spec,
            _half_spec,
            _dinv_spec,
            _w_spec,
            pl.BlockSpec((1, D), lambda i: (0, 0)),
            pl.BlockSpec((1, 1, BN), lambda i: (i, 0, 0)),
        ],
        out_specs=[
            pl.BlockSpec((BN, C), lambda i: (i, 0)),
            pl.BlockSpec((G, D), lambda i: (0, 0)),
        ],
        out_shape=[
            jax.ShapeDtypeStruct((N, C), jnp.float32),
            jax.ShapeDtypeStruct((G, D), jnp.float32),
        ],
    )
    return k1, klayer, k5


def kernel(x, edge_index, batch, W0, W1, W2, W3, fc_w, fc_b):
    pad = E_PAD - E
    src2 = jnp.concatenate(
        [edge_index[0], jnp.zeros((pad,), jnp.int32)]).reshape(NCHUNK, K)
    dst2 = jnp.concatenate(
        [edge_index[1], jnp.full((pad,), N, jnp.int32)]).reshape(NCHUNK, K)
    _deg_kernel, _scatter_kernel = _sc_kernels()
    _k1_call, _klayer_call, _k5_call = _tc_calls()

    deg = _deg_kernel(dst2)
    deg3 = deg.reshape(2, N_A, 1)
    dinv, g = _k1_call(deg3, x, W0)

    for W in (W1, W2, W3):
        A = _scatter_kernel(g, src2, dst2)
        g = _klayer_call(A, g, dinv, W)
    A = _scatter_kernel(g, src2, dst2)

    fcw_pad = jnp.zeros((D, D), jnp.float32).at[:, :C].set(fc_w)
    fcb_pad = jnp.full((1, D), -1e30, jnp.float32).at[0, :C].set(fc_b)
    batch_row = batch.reshape(GRID, 1, BN)

    logp, xr = _k5_call(A, g, dinv, fcw_pad, fcb_pad, batch_row)
    return logp, xr


# speedup vs baseline: 1.6497x; 1.0070x over previous
import functools

import jax
import jax.numpy as jnp
from jax import lax
from jax.experimental import pallas as pl
from jax.experimental.pallas import tpu as pltpu
from jax.experimental.pallas import tpu_sc as plsc

N = 10000
D = 128
H = D // 2
G = 64
C = 10
E = 320000
K = 128
CHW = 1
NB = 5
CPT = 160
NCHUNK = 16 * CPT
E_PAD = NCHUNK * K
N_A = 10240
RPT = N_A // 16
BN = 400
GRID = N // BN


def _deg_body(dst_hbm, deg_hbm, idx_v, deg_v, sum_v, slab_sh):
    c = lax.axis_index("c")
    s = lax.axis_index("s")
    wid = c * 16 + s

    def zero(i, carry):
        deg_v[pl.ds(i * 16, 16)] = jnp.zeros((16,), jnp.float32)
        return carry

    lax.fori_loop(0, N_A // 16, zero, 0)

    base = pl.multiple_of(wid * (NCHUNK // 32), 8)
    pltpu.sync_copy(dst_hbm.at[pl.ds(base, NCHUNK // 32)], idx_v)

    def row(r, carry2):
        def sub(q, carry3):
            idx = idx_v[r, pl.ds(q * 16, 16)]
            plsc.addupdate_scatter(deg_v, [idx],
                                   jnp.ones((16,), jnp.float32))
            return carry3

        return lax.fori_loop(0, K // 16, sub, carry2)

    lax.fori_loop(0, NCHUNK // 32, row, 0)

    pltpu.sync_copy(deg_v, slab_sh.at[s])
    plsc.subcore_barrier()

    row0 = pl.multiple_of(s * RPT, 8)
    pltpu.sync_copy(slab_sh.at[:, pl.ds(row0, RPT)], sum_v)

    def reduce(q, carry):
        tot = sum_v[0, pl.ds(q * 16, 16)]
        for t in range(1, 16):
            tot = tot + sum_v[t, pl.ds(q * 16, 16)]
        deg_v[pl.ds(q * 16, 16)] = tot
        return carry

    lax.fori_loop(0, RPT // 16, reduce, 0)
    pltpu.sync_copy(deg_v.at[pl.ds(0, RPT)],
                    deg_hbm.at[c, pl.ds(row0, RPT)])


def _scatter_body(g_hbm, src_hbm, dst_hbm, out_hbm, *refs):
    src_v, dst_v = refs[0], refs[1]
    rows = refs[2:2 + NB]
    acc_sh = refs[2 + NB]
    gsems = refs[3 + NB:3 + 2 * NB]
    ssems = refs[3 + 2 * NB:3 + 3 * NB]
    c = lax.axis_index("c")
    s = lax.axis_index("s")
    base = pl.multiple_of(s * CPT, 8)
    rows0_v = rows[0]

    def zero(i, carry):
        r = i // 4
        q = i - r * 4
        rows0_v[r, pl.ds(q * 16, 16)] = jnp.zeros((16,), jnp.float32)
        return carry

    lax.fori_loop(0, K * (H // 16), zero, 0)

    row0 = pl.multiple_of(s * RPT, 8)
    for kk in range(RPT // K):
        pltpu.async_copy(rows0_v, acc_sh.at[pl.ds(row0 + kk * K, K)],
                         ssems[0])
    pltpu.async_copy(src_hbm.at[pl.ds(base, CPT)], src_v, gsems[0])
    pltpu.async_copy(dst_hbm.at[pl.ds(base, CPT)], dst_v, gsems[1])
    for kk in range(RPT // K):
        pltpu.make_async_copy(rows0_v, acc_sh.at[pl.ds(row0 + kk * K, K)],
                              ssems[0]).wait()
    pltpu.make_async_copy(src_hbm.at[pl.ds(base, CPT)], src_v,
                          gsems[0]).wait()
    pltpu.make_async_copy(dst_hbm.at[pl.ds(base, CPT)], dst_v,
                          gsems[1]).wait()

    plsc.subcore_barrier()

    g_half = g_hbm.at[c]
    nstep = CPT

    def src_at(j):
        return src_v.at[j]

    def dst_at(j):
        return dst_v.at[j]

    for r in range(NB - 1):
        pltpu.async_copy(g_half.at[src_at(r)], rows[r], gsems[r])

    def ring(jj, carry):
        for r in range(NB):
            j = jj * NB + r
            rp = (r + NB - 1) % NB

            @pl.when(j > 0)
            def _():
                pltpu.make_async_copy(rows[rp], acc_sh.at[dst_at(j - 1)],
                                      ssems[rp]).wait()

            @pl.when(j + NB - 1 < nstep)
            def _():
                pltpu.async_copy(g_half.at[src_at(j + NB - 1)], rows[rp],
                                 gsems[rp])

            pltpu.make_async_copy(g_half.at[src_at(j)],
                                  rows[r], gsems[r]).wait()
            pltpu.async_copy(rows[r], acc_sh.at[dst_at(j)], ssems[r],
                             add=True)
        return carry

    lax.fori_loop(0, nstep // NB, ring, 0)
    pltpu.make_async_copy(rows[(nstep - 1) % NB],
                          acc_sh.at[dst_at(nstep - 1)],
                          ssems[(nstep - 1) % NB]).wait()

    plsc.subcore_barrier()
    pltpu.sync_copy(acc_sh.at[pl.ds(row0, RPT)],
                    out_hbm.at[c, pl.ds(row0, RPT)])


@functools.cache
def _sc_kernels():
    mesh = plsc.VectorSubcoreMesh(core_axis_name="c", subcore_axis_name="s")
    deg = pl.kernel(
        _deg_body,
        out_type=jax.ShapeDtypeStruct((2, N_A), jnp.float32),
        mesh=mesh,
        compiler_params=pltpu.CompilerParams(needs_layout_passes=False),
        scratch_types=[
            pltpu.VMEM((NCHUNK // 32, K), jnp.int32),
            pltpu.VMEM((N_A,), jnp.float32),
            pltpu.VMEM((16, RPT), jnp.float32),
            pltpu.VMEM_SHARED((16, N_A), jnp.float32),
        ],
    )
    scatter = pl.kernel(
        _scatter_body,
        out_type=jax.ShapeDtypeStruct((2, N_A, H), jnp.float32),
        mesh=mesh,
        compiler_params=pltpu.CompilerParams(use_tc_tiling_on_sc=False),
        scratch_types=[
            pltpu.VMEM((CPT, K), jnp.int32),
            pltpu.VMEM((CPT, K), jnp.int32),
        ] + [pltpu.VMEM((K, H), jnp.float32)] * NB + [
            pltpu.VMEM_SHARED((N_A, H), jnp.float32),
        ] + [pltpu.SemaphoreType.DMA] * (2 * NB),
    )
    return deg, scatter


def _k1_body(deg_ref, x_ref, w_ref, dinv_ref, g_ref):
    dinv = lax.rsqrt(1.0 + deg_ref[0] + deg_ref[1])
    dinv_ref[...] = dinv
    res = jnp.dot(x_ref[...], w_ref[...],
                  preferred_element_type=jnp.float32) * dinv
    g_ref[0] = res[:, :H]
    g_ref[1] = res[:, H:]


def _klayer_body(a_ref, g_ref, dinv_ref, w_ref, gout_ref):
    dinv = dinv_ref[...]
    pre = jnp.concatenate([a_ref[0] + g_ref[0], a_ref[1] + g_ref[1]], axis=1)
    xl = jnp.maximum(dinv * pre, 0.0)
    res = jnp.dot(xl, w_ref[...], preferred_element_type=jnp.float32) * dinv
    gout_ref[0] = res[:, :H]
    gout_ref[1] = res[:, H:]


def _k5_body(a_ref, g_ref, dinv_ref, fcw_ref, fcb_ref, batch_ref,
             logp_ref, xr_ref):
    dinv = dinv_ref[...]
    pre = jnp.concatenate([a_ref[0] + g_ref[0], a_ref[1] + g_ref[1]], axis=1)
    x4 = jnp.maximum(dinv * pre, 0.0)
    logits = jnp.dot(x4, fcw_ref[...],
                     preferred_element_type=jnp.float32) + fcb_ref[...]
    m = jnp.max(logits, axis=1, keepdims=True)
    lse = jnp.log(jnp.sum(jnp.exp(logits - m), axis=1, keepdims=True)) + m
    logp_ref[...] = (logits - lse)[:, :C]

    oh = (batch_ref[0] == lax.broadcasted_iota(jnp.int32, (G, 1), 0)
          ).astype(jnp.float32)
    contrib = jnp.dot(oh, x4, preferred_element_type=jnp.float32)

    @pl.when(pl.program_id(0) == 0)
    def _():
        xr_ref[...] = contrib

    @pl.when(pl.program_id(0) > 0)
    def _():
        xr_ref[...] = xr_ref[...] + contrib


_half_spec = pl.BlockSpec((2, BN, H), lambda i: (0, i, 0))
_dinv_spec = pl.BlockSpec((BN, 1), lambda i: (i, 0))
_w_spec = pl.BlockSpec((D, D), lambda i: (0, 0))
_g_shape = jax.ShapeDtypeStruct((2, N, H), jnp.float32)


@functools.cache
def _tc_calls(interpret=False):
    k1 = pl.pallas_call(
        _k1_body,
        interpret=interpret,
        grid=(GRID,),
        in_specs=[
            pl.BlockSpec((2, BN, 1), lambda i: (0, i, 0)),
            pl.BlockSpec((BN, D), lambda i: (i, 0)),
            _w_spec,
        ],
        out_specs=[
            _dinv_spec,
            _half_spec,
        ],
        out_shape=[
            jax.ShapeDtypeStruct((N, 1), jnp.float32),
            _g_shape,
        ],
    )

    klayer = pl.pallas_call(
        _klayer_body,
        interpret=interpret,
        grid=(GRID,),
        in_specs=[
            _half_spec,
            _half_spec,
            _dinv_spec,
            _w_spec,
        ],
        out_specs=_half_spec,
        out_shape=_g_shape,
    )

    k5 = pl.pallas_call(
        _k5_body,
        interpret=interpret,
        grid=(GRID,),
        in_specs=[
            _half_spec,
            _half_spec,
            _dinv_spec,
            _w_spec,
            pl.BlockSpec((1, D), lambda i: (0, 0)),
            pl.BlockSpec((1, 1, BN), lambda i: (i, 0, 0)),
        ],
        out_specs=[
            pl.BlockSpec((BN, C), lambda i: (i, 0)),
            pl.BlockSpec((G, D), lambda i: (0, 0)),
        ],
        out_shape=[
            jax.ShapeDtypeStruct((N, C), jnp.float32),
            jax.ShapeDtypeStruct((G, D), jnp.float32),
        ],
    )
    return k1, klayer, k5


def kernel(x, edge_index, batch, W0, W1, W2, W3, fc_w, fc_b):
    pad = E_PAD - E
    src2 = jnp.concatenate(
        [edge_index[0], jnp.zeros((pad,), jnp.int32)]).reshape(NCHUNK, K)
    dst2 = jnp.concatenate(
        [edge_index[1], jnp.full((pad,), N, jnp.int32)]).reshape(NCHUNK, K)
    _deg_kernel, _scatter_kernel = _sc_kernels()
    _k1_call, _klayer_call, _k5_call = _tc_calls()

    deg = _deg_kernel(dst2)
    deg3 = deg.reshape(2, N_A, 1)
    dinv, g = _k1_call(deg3, x, W0)

    for W in (W1, W2, W3):
        A = _scatter_kernel(g, src2, dst2)
        g = _klayer_call(A, g, dinv, W)
    A = _scatter_kernel(g, src2, dst2)

    fcw_pad = jnp.zeros((D, D), jnp.float32).at[:, :C].set(fc_w)
    fcb_pad = jnp.full((1, D), -1e30, jnp.float32).at[0, :C].set(fc_b)
    batch_row = batch.reshape(GRID, 1, BN)

    logp, xr = _k5_call(A, g, dinv, fcw_pad, fcb_pad, batch_row)
    return logp, xr


# speedup vs baseline: 1.6498x; 1.0001x over previous
import functools

import jax
import jax.numpy as jnp
from jax import lax
from jax.experimental import pallas as pl
from jax.experimental.pallas import tpu as pltpu
from jax.experimental.pallas import tpu_sc as plsc

N = 10000
D = 128
H = D // 2
G = 64
C = 10
E = 320000
K = 128
NB = 5
CPT = 160
NCHUNK = 16 * CPT
E_PAD = NCHUNK * K
N_A = 10240
RPT = N_A // 16
BN = 400
GRID = N // BN


def _deg_body(dst_hbm, deg_hbm, idx_v, deg_v, sum_v, slab_sh):
    c = lax.axis_index("c")
    s = lax.axis_index("s")
    wid = c * 16 + s

    def zero(i, carry):
        deg_v[pl.ds(i * 16, 16)] = jnp.zeros((16,), jnp.float32)
        return carry

    lax.fori_loop(0, N_A // 16, zero, 0)

    base = pl.multiple_of(wid * (NCHUNK // 32), 8)
    pltpu.sync_copy(dst_hbm.at[pl.ds(base, NCHUNK // 32)], idx_v)

    def row(r, carry2):
        def sub(q, carry3):
            idx = idx_v[r, pl.ds(q * 16, 16)]
            plsc.addupdate_scatter(deg_v, [idx],
                                   jnp.ones((16,), jnp.float32))
            return carry3

        return lax.fori_loop(0, K // 16, sub, carry2)

    lax.fori_loop(0, NCHUNK // 32, row, 0)

    pltpu.sync_copy(deg_v, slab_sh.at[s])
    plsc.subcore_barrier()

    row0 = pl.multiple_of(s * RPT, 8)
    pltpu.sync_copy(slab_sh.at[:, pl.ds(row0, RPT)], sum_v)

    def reduce(q, carry):
        tot = sum_v[0, pl.ds(q * 16, 16)]
        for t in range(1, 16):
            tot = tot + sum_v[t, pl.ds(q * 16, 16)]
        deg_v[pl.ds(q * 16, 16)] = tot
        return carry

    lax.fori_loop(0, RPT // 16, reduce, 0)
    pltpu.sync_copy(deg_v.at[pl.ds(0, RPT)],
                    deg_hbm.at[c, pl.ds(row0, RPT)])


def _scatter_body(g_hbm, src_hbm, dst_hbm, out_hbm, *refs):
    src_v, dst_v = refs[0], refs[1]
    rows = refs[2:2 + NB]
    acc_sh = refs[2 + NB]
    gsems = refs[3 + NB:3 + 2 * NB]
    ssems = refs[3 + 2 * NB:3 + 3 * NB]
    c = lax.axis_index("c")
    s = lax.axis_index("s")
    base = pl.multiple_of(s * CPT, 8)
    rows0_v = rows[0]

    def zero(i, carry):
        r = i // 4
        q = i - r * 4
        rows0_v[r, pl.ds(q * 16, 16)] = jnp.zeros((16,), jnp.float32)
        return carry

    lax.fori_loop(0, K * (H // 16), zero, 0)

    row0 = pl.multiple_of(s * RPT, 8)
    for kk in range(RPT // K):
        pltpu.async_copy(rows0_v, acc_sh.at[pl.ds(row0 + kk * K, K)],
                         ssems[0])
    pltpu.async_copy(src_hbm.at[pl.ds(base, CPT)], src_v, gsems[0])
    pltpu.async_copy(dst_hbm.at[pl.ds(base, CPT)], dst_v, gsems[1])
    for kk in range(RPT // K):
        pltpu.make_async_copy(rows0_v, acc_sh.at[pl.ds(row0 + kk * K, K)],
                              ssems[0]).wait()
    pltpu.make_async_copy(src_hbm.at[pl.ds(base, CPT)], src_v,
                          gsems[0]).wait()
    pltpu.make_async_copy(dst_hbm.at[pl.ds(base, CPT)], dst_v,
                          gsems[1]).wait()

    plsc.subcore_barrier()

    g_half = g_hbm.at[c]
    nstep = CPT

    def src_at(j):
        return src_v.at[j]

    def dst_at(j):
        return dst_v.at[j]

    for r in range(NB - 1):
        pltpu.async_copy(g_half.at[src_at(r)], rows[r], gsems[r])

    def ring(jj, carry):
        for r in range(NB):
            j = jj * NB + r
            rp = (r + NB - 1) % NB

            @pl.when(j > 0)
            def _():
                pltpu.make_async_copy(rows[rp], acc_sh.at[dst_at(j - 1)],
                                      ssems[rp]).wait()

            @pl.when(j + NB - 1 < nstep)
            def _():
                pltpu.async_copy(g_half.at[src_at(j + NB - 1)], rows[rp],
                                 gsems[rp])

            pltpu.make_async_copy(g_half.at[src_at(j)],
                                  rows[r], gsems[r]).wait()
            pltpu.async_copy(rows[r], acc_sh.at[dst_at(j)], ssems[r],
                             add=True)
        return carry

    lax.fori_loop(0, nstep // NB, ring, 0)
    pltpu.make_async_copy(rows[(nstep - 1) % NB],
                          acc_sh.at[dst_at(nstep - 1)],
                          ssems[(nstep - 1) % NB]).wait()

    plsc.subcore_barrier()
    pltpu.sync_copy(acc_sh.at[pl.ds(row0, RPT)],
                    out_hbm.at[c, pl.ds(row0, RPT)])


@functools.cache
def _sc_kernels():
    mesh = plsc.VectorSubcoreMesh(core_axis_name="c", subcore_axis_name="s")
    deg = pl.kernel(
        _deg_body,
        out_type=jax.ShapeDtypeStruct((2, N_A), jnp.float32),
        mesh=mesh,
        compiler_params=pltpu.CompilerParams(needs_layout_passes=False),
        scratch_types=[
            pltpu.VMEM((NCHUNK // 32, K), jnp.int32),
            pltpu.VMEM((N_A,), jnp.float32),
            pltpu.VMEM((16, RPT), jnp.float32),
            pltpu.VMEM_SHARED((16, N_A), jnp.float32),
        ],
    )
    scatter = pl.kernel(
        _scatter_body,
        out_type=jax.ShapeDtypeStruct((2, N_A, H), jnp.float32),
        mesh=mesh,
        compiler_params=pltpu.CompilerParams(use_tc_tiling_on_sc=False),
        scratch_types=[
            pltpu.VMEM((CPT, K), jnp.int32),
            pltpu.VMEM((CPT, K), jnp.int32),
        ] + [pltpu.VMEM((K, H), jnp.float32)] * NB + [
            pltpu.VMEM_SHARED((N_A, H), jnp.float32),
        ] + [pltpu.SemaphoreType.DMA] * (2 * NB),
    )
    return deg, scatter


def _k1_body(deg_ref, x_ref, w_ref, dinv_ref, g_ref):
    dinv = lax.rsqrt(1.0 + deg_ref[0] + deg_ref[1])
    dinv_ref[...] = dinv
    res = jnp.dot(x_ref[...], w_ref[...],
                  preferred_element_type=jnp.float32) * dinv
    g_ref[0] = res[:, :H]
    g_ref[1] = res[:, H:]


def _klayer_body(a_ref, g_ref, dinv_ref, w_ref, gout_ref):
    dinv = dinv_ref[...]
    pre = jnp.concatenate([a_ref[0] + g_ref[0], a_ref[1] + g_ref[1]], axis=1)
    xl = jnp.maximum(dinv * pre, 0.0)
    res = jnp.dot(xl, w_ref[...], preferred_element_type=jnp.float32) * dinv
    gout_ref[0] = res[:, :H]
    gout_ref[1] = res[:, H:]


def _k5_body(a_ref, g_ref, dinv_ref, fcw_ref, fcb_ref, batch_ref,
             logp_ref, xr_ref):
    dinv = dinv_ref[...]
    pre = jnp.concatenate([a_ref[0] + g_ref[0], a_ref[1] + g_ref[1]], axis=1)
    x4 = jnp.maximum(dinv * pre, 0.0)
    logits = jnp.dot(x4, fcw_ref[...],
                     preferred_element_type=jnp.float32) + fcb_ref[...]
    m = jnp.max(logits, axis=1, keepdims=True)
    lse = jnp.log(jnp.sum(jnp.exp(logits - m), axis=1, keepdims=True)) + m
    logp_ref[...] = (logits - lse)[:, :C]

    oh = (batch_ref[0] == lax.broadcasted_iota(jnp.int32, (G, 1), 0)
          ).astype(jnp.float32)
    contrib = jnp.dot(oh, x4, preferred_element_type=jnp.float32)

    @pl.when(pl.program_id(0) == 0)
    def _():
        xr_ref[...] = contrib

    @pl.when(pl.program_id(0) > 0)
    def _():
        xr_ref[...] = xr_ref[...] + contrib


_half_spec = pl.BlockSpec((2, BN, H), lambda i: (0, i, 0))
_dinv_spec = pl.BlockSpec((BN, 1), lambda i: (i, 0))
_w_spec = pl.BlockSpec((D, D), lambda i: (0, 0))
_g_shape = jax.ShapeDtypeStruct((2, N, H), jnp.float32)


@functools.cache
def _tc_calls(interpret=False):
    k1 = pl.pallas_call(
        _k1_body,
        interpret=interpret,
        grid=(GRID,),
        in_specs=[
            pl.BlockSpec((2, BN, 1), lambda i: (0, i, 0)),
            pl.BlockSpec((BN, D), lambda i: (i, 0)),
            _w_spec,
        ],
        out_specs=[
            _dinv_spec,
            _half_spec,
        ],
        out_shape=[
            jax.ShapeDtypeStruct((N, 1), jnp.float32),
            _g_shape,
        ],
    )

    klayer = pl.pallas_call(
        _klayer_body,
        interpret=interpret,
        grid=(GRID,),
        in_specs=[
            _half_spec,
            _half_spec,
            _dinv_spec,
            _w_spec,
        ],
        out_specs=_half_spec,
        out_shape=_g_shape,
    )

    k5 = pl.pallas_call(
        _k5_body,
        interpret=interpret,
        grid=(GRID,),
        in_specs=[
            _half_spec,
            _half_spec,
            _dinv_spec,
            _w_spec,
            pl.BlockSpec((1, D), lambda i: (0, 0)),
            pl.BlockSpec((1, 1, BN), lambda i: (i, 0, 0)),
        ],
        out_specs=[
            pl.BlockSpec((BN, C), lambda i: (i, 0)),
            pl.BlockSpec((G, D), lambda i: (0, 0)),
        ],
        out_shape=[
            jax.ShapeDtypeStruct((N, C), jnp.float32),
            jax.ShapeDtypeStruct((G, D), jnp.float32),
        ],
    )
    return k1, klayer, k5


def kernel(x, edge_index, batch, W0, W1, W2, W3, fc_w, fc_b):
    pad = E_PAD - E
    src2 = jnp.concatenate(
        [edge_index[0], jnp.zeros((pad,), jnp.int32)]).reshape(NCHUNK, K)
    dst2 = jnp.concatenate(
        [edge_index[1], jnp.full((pad,), N, jnp.int32)]).reshape(NCHUNK, K)
    _deg_kernel, _scatter_kernel = _sc_kernels()
    _k1_call, _klayer_call, _k5_call = _tc_calls()

    deg = _deg_kernel(dst2)
    deg3 = deg.reshape(2, N_A, 1)
    dinv, g = _k1_call(deg3, x, W0)

    for W in (W1, W2, W3):
        A = _scatter_kernel(g, src2, dst2)
        g = _klayer_call(A, g, dinv, W)
    A = _scatter_kernel(g, src2, dst2)

    fcw_pad = jnp.zeros((D, D), jnp.float32).at[:, :C].set(fc_w)
    fcb_pad = jnp.full((1, D), -1e30, jnp.float32).at[0, :C].set(fc_b)
    batch_row = batch.reshape(GRID, 1, BN)

    logp, xr = _k5_call(A, g, dinv, fcw_pad, fcb_pad, batch_row)
    return logp, xr
